# baseline (speedup 1.0000x reference)
import jax
import jax.numpy as jnp
from jax import lax
from jax.experimental import pallas as pl
from jax.experimental.pallas import tpu as pltpu
from jax.experimental.pallas import tpu_sc as plsc

N_ATOMS = 500000
N_TYPES = 100
N_PROPS = 128
N_SYSTEMS = 2048

NC = 2
NS = 16
LANES = 16
NW = NC * NS

STRIDE = 15584
CHUNK = 16896
NROWS = CHUNK // 128
NVREG = CHUNK // LANES
NOWNED = STRIDE // LANES
HBINS = N_SYSTEMS * 128
ZSLICE = HBINS // NS
DEAD_BIN = 127

assert (NW - 1) * STRIDE + CHUNK == N_ATOMS
assert STRIDE % LANES == 0 and CHUNK % 128 == 0


def _sc_hist_body(types_hbm, sys_hbm, zeros_hbm, ones_hbm, out_hbm,
                  types_v, sys_v, idx_v, val_v, buf_v, shared,
                  sem_t, sem_s, sem_o):
    c = lax.axis_index("c")
    s = lax.axis_index("s")
    wid = c * NS + s
    base = wid * STRIDE

    cp_t = pltpu.async_copy(types_hbm.at[pl.ds(base, CHUNK)], types_v, sem_t)
    cp_s = pltpu.async_copy(sys_hbm.at[pl.ds(base, CHUNK)], sys_v, sem_s)
    cp_o = pltpu.async_copy(ones_hbm, val_v, sem_o)
    pltpu.sync_copy(zeros_hbm.at[pl.ds(s * ZSLICE, ZSLICE)],
                    shared.at[pl.ds(s * ZSLICE, ZSLICE)])
    cp_t.wait()
    cp_s.wait()
    cp_o.wait()

    def comp_body(i):
        t = types_v[pl.ds(i * LANES, LANES)]
        sy = sys_v[pl.ds(i * LANES, LANES)]
        comb = sy * 128 + t
        idx_v[i // 8, pl.ds((i % 8) * LANES, LANES)] = comb

    plsc.parallel_loop(0, NOWNED, unroll=8)(comp_body)

    def tail_body(i):
        t = types_v[pl.ds(i * LANES, LANES)]
        sy = sys_v[pl.ds(i * LANES, LANES)]
        keep = (wid == NW - 1).astype(jnp.int32)
        comb = keep * (sy * 128 + t) + (1 - keep) * DEAD_BIN
        idx_v[i // 8, pl.ds((i % 8) * LANES, LANES)] = comb

    plsc.parallel_loop(NOWNED, NVREG, unroll=2)(tail_body)

    plsc.subcore_barrier()

    DEPTH = 32

    def scat_body(j, _):
        pltpu.make_async_copy(
            val_v.at[j], shared.at[idx_v.at[j]], sem_t).start(add=True)
        @pl.when(j >= DEPTH)
        def _wait():
            pltpu.make_async_copy(
                val_v.at[j - DEPTH], shared.at[idx_v.at[j - DEPTH]],
                sem_t).wait()
        return _

    lax.fori_loop(0, NROWS, scat_body, None)

    def drain_body(j, _):
        pltpu.make_async_copy(
            val_v.at[j], shared.at[idx_v.at[j]], sem_t).wait()
        return _

    lax.fori_loop(NROWS - DEPTH, NROWS, drain_body, None)

    plsc.subcore_barrier()

    pltpu.sync_copy(shared.at[pl.ds(s * ZSLICE, ZSLICE)], buf_v)
    pltpu.sync_copy(buf_v, out_hbm.at[c, pl.ds(s * ZSLICE, ZSLICE)])


def _sc_hist(atom_types, system_indices):
    mesh = plsc.VectorSubcoreMesh(core_axis_name="c", subcore_axis_name="s")
    zeros = jnp.zeros((HBINS,), jnp.float32)
    ones = jnp.ones((NROWS, 128), jnp.float32)
    return pl.kernel(
        _sc_hist_body,
        out_type=jax.ShapeDtypeStruct((NC, HBINS), jnp.float32),
        mesh=mesh,
        scratch_types=[
            pltpu.VMEM((CHUNK,), jnp.int32),
            pltpu.VMEM((CHUNK,), jnp.int32),
            pltpu.VMEM((NROWS, 128), jnp.int32),
            pltpu.VMEM((NROWS, 128), jnp.float32),
            pltpu.VMEM((ZSLICE,), jnp.float32),
            pltpu.VMEM_SHARED((HBINS,), jnp.float32),
            pltpu.SemaphoreType.DMA,
            pltpu.SemaphoreType.DMA,
            pltpu.SemaphoreType.DMA,
        ],
    )(atom_types, system_indices, zeros, ones)


def _tc_matmul_body(hist_ref, w_ref, t2i_ref, out_ref):
    h = hist_ref[0] + hist_ref[1]
    r = lax.broadcasted_iota(jnp.int32, (128, 128), 1)
    m = (t2i_ref[...] == r).astype(jnp.float32)
    w_eff = jnp.dot(m, w_ref[...], preferred_element_type=jnp.float32)
    out_ref[...] = jnp.dot(h, w_eff, preferred_element_type=jnp.float32)


def _tc_matmul(hist, w_pad, t2i_pad):
    return pl.pallas_call(
        _tc_matmul_body,
        out_shape=jax.ShapeDtypeStruct((N_SYSTEMS, N_PROPS), jnp.float32),
        in_specs=[
            pl.BlockSpec(memory_space=pltpu.VMEM),
            pl.BlockSpec(memory_space=pltpu.VMEM),
            pl.BlockSpec(memory_space=pltpu.VMEM),
        ],
        out_specs=pl.BlockSpec(memory_space=pltpu.VMEM),
    )(hist, w_pad, t2i_pad)


def kernel(atom_types, system_indices, weights, type_to_index):
    hist = _sc_hist(atom_types, system_indices)
    hist = hist.reshape(NC, N_SYSTEMS, 128)
    w_pad = jnp.pad(weights, ((0, 128 - N_TYPES), (0, 0)))
    t2i_pad = jnp.pad(type_to_index, (0, 128 - N_TYPES),
                      constant_values=127).reshape(128, 1)
    return _tc_matmul(hist, w_pad, t2i_pad)
